# baseline (speedup 1.0000x reference)
import functools

import jax
import jax.numpy as jnp
from jax import lax
from jax.experimental import pallas as pl
from jax.experimental.pallas import tpu as pltpu
from jax.experimental.pallas import tpu_sc as plsc

N_NODES = 10000
N_EDGES = 320000
IN_DIM = 128
HID = 16

NC = 2
NS = 16
NW = NC * NS

CHUNK = 1024
EPW = N_EDGES // NW
CPW = 10
NBUF = 4
NBUF1 = 3
SLAB_C = CPW
E_PAD = NW * SLAB_C * CHUNK

ACC_ROWS = 10240
STRIP = ACC_ROWS // NS
DUMMY = N_NODES

_mesh = plsc.VectorSubcoreMesh(core_axis_name="c", subcore_axis_name="s")


def _edge_ring(table_sh, acc_sh, src_idx, dst_idx, bufs, semg, sems, nbuf):
    lead = nbuf - 2
    gathers = [None] * CPW
    scatters = [None] * CPW
    for c in range(lead):
        gathers[c] = pltpu.async_copy(
            table_sh.at[src_idx(c)], bufs[c % nbuf], semg[c])
    for c in range(CPW):
        gathers[c].wait()
        scatters[c] = pltpu.async_copy(
            bufs[c % nbuf], acc_sh.at[dst_idx(c)], sems[c], add=True)
        j = c + lead
        if j < CPW:
            if j - nbuf >= 0:
                scatters[j - nbuf].wait()
            gathers[j] = pltpu.async_copy(
                table_sh.at[src_idx(j)], bufs[j % nbuf], semg[j])
    for c in range(CPW - nbuf, CPW):
        if scatters[c] is not None:
            scatters[c].wait()


def _fast_rsqrt(deg):
    y = deg
    for _ in range(18):
        y = 0.5 * (y + deg / y)
    return 1.0 / y


@functools.partial(
    pl.kernel,
    out_type=(
        jax.ShapeDtypeStruct((NC, ACC_ROWS, HID), jnp.float32),
        jax.ShapeDtypeStruct((ACC_ROWS, HID), jnp.float32),
    ),
    mesh=_mesh,
    scratch_types=[
        pltpu.VMEM((SLAB_C, CHUNK), jnp.int32),
        pltpu.VMEM((2, SLAB_C, CHUNK), jnp.int32),
        pltpu.VMEM((STRIP, HID), jnp.float32),
        pltpu.VMEM((STRIP, HID), jnp.float32),
        pltpu.VMEM((STRIP,), jnp.float32),
        pltpu.VMEM((CHUNK,), jnp.float32),
        pltpu.VMEM_SHARED((ACC_ROWS,), jnp.float32),
        pltpu.VMEM_SHARED((ACC_ROWS, HID), jnp.float32),
        pltpu.VMEM_SHARED((ACC_ROWS, HID), jnp.float32),
    ] + [pltpu.VMEM((CHUNK, HID), jnp.float32)] * NBUF1
      + [pltpu.SemaphoreType.DMA] * (2 * CPW),
    compiler_params=pltpu.CompilerParams(use_tc_tiling_on_sc=False),
)
def _sc_layer1(h1_hbm, src_hbm, dst_hbm, part_hbm, dinv_hbm,
               src_v, dst2_v, h_v, dinv_v, cnt_v, ones_v,
               cnt_sh, table_sh, acc_sh, *rest):
    bufs = rest[:NBUF1]
    semg = rest[NBUF1:NBUF1 + CPW]
    sems = rest[NBUF1 + CPW:]
    cid = lax.axis_index("c")
    sid = lax.axis_index("s")
    wid = sid * NC + cid
    strip = pl.ds(sid * STRIP, STRIP)

    zero16 = jnp.zeros((16,), jnp.float32)
    ones16 = jnp.ones((16,), jnp.float32)

    def zero_cnt(i, carry):
        cnt_v[pl.ds(i * 16, 16)] = zero16
        return carry

    lax.fori_loop(0, STRIP // 16, zero_cnt, 0)
    pltpu.sync_copy(cnt_v, cnt_sh.at[strip])
    for i in range(CHUNK // 16):
        ones_v[pl.ds(i * 16, 16)] = ones16
    pltpu.sync_copy(h1_hbm.at[strip], h_v)
    pltpu.sync_copy(src_hbm.at[wid], src_v)
    pltpu.sync_copy(dst_hbm.at[pl.ds(2 * sid, 2)], dst2_v)

    def zero_buf(i, carry):
        bufs[0][i, :] = zero16
        return carry

    lax.fori_loop(0, CHUNK, zero_buf, 0)
    plsc.subcore_barrier()

    all_sems = list(semg) + list(sems)
    cnt_copies = [
        pltpu.async_copy(
            ones_v, cnt_sh.at[dst2_v.at[a, c]], all_sems[a * CPW + c], add=True)
        for a in range(2) for c in range(CPW)
    ]
    for cp in cnt_copies:
        cp.wait()
    done = 0
    while done < STRIP:
        step = min(CHUNK, STRIP - done)
        pltpu.sync_copy(
            bufs[0].at[pl.ds(0, step)],
            acc_sh.at[pl.ds(sid * STRIP + done, step)],
        )
        done += step
    plsc.subcore_barrier()

    pltpu.sync_copy(cnt_sh.at[strip], cnt_v)

    def row_body(k, carry):
        y16 = _fast_rsqrt(cnt_v[pl.ds(k * 16, 16)] + 1.0)
        for l in range(16):
            r = k * 16 + l
            yl = jnp.full((16,), y16[l], jnp.float32)
            h_v[r, :] = h_v[r, :] * yl
            dinv_v[r, :] = yl
        return carry

    lax.fori_loop(0, STRIP // 16, row_body, 0)
    pltpu.sync_copy(h_v, table_sh.at[strip])

    @pl.when(cid == 0)
    def _():
        pltpu.sync_copy(dinv_v, dinv_hbm.at[strip])

    plsc.subcore_barrier()

    _edge_ring(
        table_sh, acc_sh,
        lambda c: src_v.at[c],
        lambda c: dst2_v.at[cid, c],
        bufs, semg, sems, NBUF1)
    plsc.subcore_barrier()
    pltpu.sync_copy(acc_sh.at[strip], part_hbm.at[cid, strip])


@functools.partial(
    pl.kernel,
    out_type=jax.ShapeDtypeStruct((NC, ACC_ROWS, HID), jnp.float32),
    mesh=_mesh,
    scratch_types=[
        pltpu.VMEM((SLAB_C, CHUNK), jnp.int32),
        pltpu.VMEM((SLAB_C, CHUNK), jnp.int32),
    ] + [pltpu.VMEM((CHUNK, HID), jnp.float32)] * NBUF
      + [pltpu.VMEM_SHARED((ACC_ROWS, HID), jnp.float32)]
      + [pltpu.VMEM_SHARED((ACC_ROWS, HID), jnp.float32)]
      + [pltpu.SemaphoreType.DMA] * (2 * CPW),
    compiler_params=pltpu.CompilerParams(use_tc_tiling_on_sc=False),
)
def _sc_scatter_rows(vals_hbm, src_hbm, dst_hbm, out_hbm,
                     src_v, dst_v, *rest):
    bufs = rest[:NBUF]
    acc_sh = rest[NBUF]
    table_sh = rest[NBUF + 1]
    semg = rest[NBUF + 2:NBUF + 2 + CPW]
    sems = rest[NBUF + 2 + CPW:]
    cid = lax.axis_index("c")
    sid = lax.axis_index("s")
    wid = sid * NC + cid
    strip = pl.ds(sid * STRIP, STRIP)

    pltpu.sync_copy(vals_hbm.at[strip], table_sh.at[strip])
    zero16 = jnp.zeros((16,), jnp.float32)

    def zero_buf(i, carry):
        bufs[0][i, :] = zero16
        return carry

    lax.fori_loop(0, CHUNK, zero_buf, 0)
    done = 0
    while done < STRIP:
        step = min(CHUNK, STRIP - done)
        pltpu.sync_copy(
            bufs[0].at[pl.ds(0, step)],
            acc_sh.at[pl.ds(sid * STRIP + done, step)],
        )
        done += step
    pltpu.sync_copy(src_hbm.at[wid], src_v)
    pltpu.sync_copy(dst_hbm.at[wid], dst_v)
    plsc.subcore_barrier()

    _edge_ring(
        table_sh, acc_sh,
        lambda c: src_v.at[c],
        lambda c: dst_v.at[c],
        bufs, semg, sems, NBUF)
    plsc.subcore_barrier()
    pltpu.sync_copy(acc_sh.at[strip], out_hbm.at[cid, strip])


def _tc1_body(x_ref, w1_ref, h1_ref):
    h1_ref[...] = jnp.dot(
        x_ref[...], w1_ref[...], preferred_element_type=jnp.float32)


def _tc2_body(p0_ref, p1_ref, h1_ref, dinv_ref, w2_ref, b1_ref, g2_ref):
    dinv = dinv_ref[...]
    s1 = p0_ref[...] + p1_ref[...] + dinv * h1_ref[...]
    out1 = jnp.maximum(s1 * dinv + b1_ref[...], 0.0)
    h2 = jnp.dot(out1, w2_ref[...], preferred_element_type=jnp.float32)
    g2_ref[...] = h2 * dinv


def _tc3_body(p0_ref, p1_ref, g2_ref, dinv_ref, b2_ref, wfc_ref, bfc_ref,
              o_ref):
    dinv = dinv_ref[...]
    s2 = p0_ref[...] + p1_ref[...] + g2_ref[...]
    out2 = jnp.maximum(s2 * dinv + b2_ref[...], 0.0)
    pooled = jnp.sum(out2, axis=0, keepdims=True) * (1.0 / N_NODES)
    z = jnp.dot(pooled, wfc_ref[...], preferred_element_type=jnp.float32)
    o_ref[...] = jax.nn.sigmoid(z + bfc_ref[...])


def kernel(x, edge_index, W1, b1, W2, b2, Wfc, bfc):
    pad = SLAB_C * CHUNK - EPW
    src3 = jnp.pad(
        edge_index[0].astype(jnp.int32).reshape(NW, EPW), ((0, 0), (0, pad)),
    ).reshape(NW, SLAB_C, CHUNK)
    dst3 = jnp.pad(
        edge_index[1].astype(jnp.int32).reshape(NW, EPW), ((0, 0), (0, pad)),
        constant_values=DUMMY,
    ).reshape(NW, SLAB_C, CHUNK)

    h1 = pl.pallas_call(
        _tc1_body,
        out_shape=jax.ShapeDtypeStruct((N_NODES, HID), jnp.float32),
    )(x, W1)
    h1p = jnp.pad(h1, ((0, ACC_ROWS - N_NODES), (0, 0)))

    p1, dinv_rep = _sc_layer1(h1p, src3, dst3)
    dinv2 = dinv_rep[:N_NODES]

    g2 = pl.pallas_call(
        _tc2_body,
        out_shape=jax.ShapeDtypeStruct((N_NODES, HID), jnp.float32),
    )(p1[0, :N_NODES], p1[1, :N_NODES], h1, dinv2, W2, b1.reshape(1, HID))

    p2 = _sc_scatter_rows(
        jnp.pad(g2, ((0, ACC_ROWS - N_NODES), (0, 0))), src3, dst3)

    out = pl.pallas_call(
        _tc3_body,
        out_shape=jax.ShapeDtypeStruct((1, 1), jnp.float32),
    )(p2[0, :N_NODES], p2[1, :N_NODES], g2, dinv2, b2.reshape(1, HID),
      Wfc, bfc.reshape(1, 1))
    return out.reshape(1)


# speedup vs baseline: 1.0600x; 1.0600x over previous
import functools

import jax
import jax.numpy as jnp
from jax import lax
from jax.experimental import pallas as pl
from jax.experimental.pallas import tpu as pltpu
from jax.experimental.pallas import tpu_sc as plsc

N_NODES = 10000
N_EDGES = 320000
IN_DIM = 128
HID = 16

NC = 2
NS = 16
NW = NC * NS

EPW = N_EDGES // NW
CHUNK = 1000
CPW = EPW // CHUNK
NBUF = 4

ACC_ROWS = 10240
STRIP = ACC_ROWS // NS

_mesh = plsc.VectorSubcoreMesh(core_axis_name="c", subcore_axis_name="s")


@functools.partial(
    pl.kernel,
    out_type=jax.ShapeDtypeStruct((NC, ACC_ROWS, HID), jnp.float32),
    mesh=_mesh,
    scratch_types=[
        pltpu.VMEM((CPW, CHUNK), jnp.int32),
        pltpu.VMEM((CHUNK, HID), jnp.float32),
        pltpu.VMEM_SHARED((ACC_ROWS, HID), jnp.float32),
    ] + [pltpu.SemaphoreType.DMA] * CPW,
    compiler_params=pltpu.CompilerParams(use_tc_tiling_on_sc=False),
)
def _sc_counts(edges_hbm, out_hbm, dst_v, buf, acc_sh, *sems):
    cid = lax.axis_index("c")
    sid = lax.axis_index("s")
    wid = sid * NC + cid
    strip = pl.ds(sid * STRIP, STRIP)
    base = wid * EPW

    zero16 = jnp.zeros((16,), jnp.float32)

    def zero_buf(i, carry):
        buf[i, :] = zero16
        return carry

    lax.fori_loop(0, CHUNK, zero_buf, 0)
    pltpu.sync_copy(buf.at[pl.ds(0, STRIP)], acc_sh.at[strip])
    ones16 = jnp.ones((16,), jnp.float32)

    def ones_buf(i, carry):
        buf[i, :] = ones16
        return carry

    lax.fori_loop(0, CHUNK, ones_buf, 0)
    for c in range(CPW):
        pltpu.sync_copy(
            edges_hbm.at[1, pl.ds(base + c * CHUNK, CHUNK)], dst_v.at[c])
    plsc.subcore_barrier()

    copies = [
        pltpu.async_copy(buf, acc_sh.at[dst_v.at[c]], sems[c], add=True)
        for c in range(CPW)
    ]
    for cp in copies:
        cp.wait()
    plsc.subcore_barrier()
    pltpu.sync_copy(acc_sh.at[strip], out_hbm.at[cid, strip])


@functools.partial(
    pl.kernel,
    out_type=jax.ShapeDtypeStruct((NC, ACC_ROWS, HID), jnp.float32),
    mesh=_mesh,
    scratch_types=[
        pltpu.VMEM((CPW, CHUNK), jnp.int32),
        pltpu.VMEM((CPW, CHUNK), jnp.int32),
    ] + [pltpu.VMEM((CHUNK, HID), jnp.float32)] * NBUF
      + [pltpu.VMEM_SHARED((ACC_ROWS, HID), jnp.float32)]
      + [pltpu.VMEM_SHARED((ACC_ROWS, HID), jnp.float32)]
      + [pltpu.SemaphoreType.DMA] * (2 * CPW),
    compiler_params=pltpu.CompilerParams(use_tc_tiling_on_sc=False),
)
def _sc_scatter_rows(vals_hbm, edges_hbm, out_hbm, src_v, dst_v, *rest):
    bufs = rest[:NBUF]
    acc_sh = rest[NBUF]
    table_sh = rest[NBUF + 1]
    semg = rest[NBUF + 2:NBUF + 2 + CPW]
    sems = rest[NBUF + 2 + CPW:]
    cid = lax.axis_index("c")
    sid = lax.axis_index("s")
    wid = sid * NC + cid
    strip = pl.ds(sid * STRIP, STRIP)
    base = wid * EPW

    pltpu.sync_copy(vals_hbm.at[strip], table_sh.at[strip])
    zero16 = jnp.zeros((16,), jnp.float32)

    def zero_buf(i, carry):
        bufs[0][i, :] = zero16
        return carry

    lax.fori_loop(0, CHUNK, zero_buf, 0)
    pltpu.sync_copy(bufs[0].at[pl.ds(0, STRIP)], acc_sh.at[strip])
    for c in range(CPW):
        pltpu.sync_copy(
            edges_hbm.at[0, pl.ds(base + c * CHUNK, CHUNK)], src_v.at[c])
        pltpu.sync_copy(
            edges_hbm.at[1, pl.ds(base + c * CHUNK, CHUNK)], dst_v.at[c])
    plsc.subcore_barrier()

    LEAD = NBUF - 2
    gathers = [None] * CPW
    scatters = [None] * CPW
    for c in range(LEAD):
        gathers[c] = pltpu.async_copy(
            table_sh.at[src_v.at[c]], bufs[c % NBUF], semg[c])
    for c in range(CPW):
        gathers[c].wait()
        scatters[c] = pltpu.async_copy(
            bufs[c % NBUF], acc_sh.at[dst_v.at[c]], sems[c], add=True)
        j = c + LEAD
        if j < CPW:
            if j - NBUF >= 0:
                scatters[j - NBUF].wait()
            gathers[j] = pltpu.async_copy(
                table_sh.at[src_v.at[j]], bufs[j % NBUF], semg[j])
    for c in range(CPW - NBUF, CPW):
        if scatters[c] is not None:
            scatters[c].wait()
    plsc.subcore_barrier()
    pltpu.sync_copy(acc_sh.at[strip], out_hbm.at[cid, strip])


def _tc1_body(c_ref, x_ref, w1_ref, g1_ref, dinv_ref):
    dinv = lax.rsqrt(c_ref[0] + c_ref[1] + 1.0)
    dinv_ref[...] = dinv
    h = jnp.dot(x_ref[...], w1_ref[...], preferred_element_type=jnp.float32)
    g1_ref[pl.ds(0, N_NODES), :] = h * dinv[:N_NODES]
    g1_ref[pl.ds(N_NODES, ACC_ROWS - N_NODES), :] = jnp.zeros(
        (ACC_ROWS - N_NODES, HID), jnp.float32)


def _tc2_body(p_ref, g1_ref, dinv_ref, w2_ref, b1_ref, g2_ref):
    dinv = dinv_ref[...]
    s1 = (p_ref[0] + p_ref[1] + g1_ref[...])[:N_NODES]
    out1 = jnp.maximum(s1 * dinv[:N_NODES] + b1_ref[...], 0.0)
    h2 = jnp.dot(out1, w2_ref[...], preferred_element_type=jnp.float32)
    g2_ref[pl.ds(0, N_NODES), :] = h2 * dinv[:N_NODES]
    g2_ref[pl.ds(N_NODES, ACC_ROWS - N_NODES), :] = jnp.zeros(
        (ACC_ROWS - N_NODES, HID), jnp.float32)


def _tc3_body(p_ref, g2_ref, dinv_ref, b2_ref, wfc_ref, bfc_ref, o_ref):
    s2 = (p_ref[0] + p_ref[1] + g2_ref[...])[:N_NODES]
    out2 = jnp.maximum(s2 * dinv_ref[pl.ds(0, N_NODES), :] + b2_ref[...], 0.0)
    pooled = jnp.sum(out2, axis=0, keepdims=True) * (1.0 / N_NODES)
    z = jnp.dot(pooled, wfc_ref[...], preferred_element_type=jnp.float32)
    o_ref[...] = jax.nn.sigmoid(z + bfc_ref[...])


def kernel(x, edge_index, W1, b1, W2, b2, Wfc, bfc):
    edges = edge_index.astype(jnp.int32)

    cnt = _sc_counts(edges)

    g1, dinv = pl.pallas_call(
        _tc1_body,
        out_shape=(
            jax.ShapeDtypeStruct((ACC_ROWS, HID), jnp.float32),
            jax.ShapeDtypeStruct((ACC_ROWS, HID), jnp.float32),
        ),
    )(cnt, x, W1)

    p1 = _sc_scatter_rows(g1, edges)

    g2 = pl.pallas_call(
        _tc2_body,
        out_shape=jax.ShapeDtypeStruct((ACC_ROWS, HID), jnp.float32),
    )(p1, g1, dinv, W2, b1.reshape(1, HID))

    p2 = _sc_scatter_rows(g2, edges)

    out = pl.pallas_call(
        _tc3_body,
        out_shape=jax.ShapeDtypeStruct((1, 1), jnp.float32),
    )(p2, g2, dinv, b2.reshape(1, HID), Wfc, bfc.reshape(1, 1))
    return out.reshape(1)


# speedup vs baseline: 1.1466x; 1.0818x over previous
import functools

import jax
import jax.numpy as jnp
from jax import lax
from jax.experimental import pallas as pl
from jax.experimental.pallas import tpu as pltpu
from jax.experimental.pallas import tpu_sc as plsc

N_NODES = 10000
N_EDGES = 320000
IN_DIM = 128
HID = 16

NC = 2
NS = 16
NW = NC * NS

EPW = N_EDGES // NW
CHUNK = 1000
CPW = EPW // CHUNK
NBUF = 4

ACC_ROWS = 10240
STRIP = ACC_ROWS // NS

_mesh = plsc.VectorSubcoreMesh(core_axis_name="c", subcore_axis_name="s")


@functools.partial(
    pl.kernel,
    out_type=jax.ShapeDtypeStruct((NC, ACC_ROWS, HID), jnp.float32),
    mesh=_mesh,
    scratch_types=[
        pltpu.VMEM((CPW, CHUNK), jnp.int32),
        pltpu.VMEM((CHUNK,), jnp.float32),
        pltpu.VMEM((STRIP, HID), jnp.float32),
        pltpu.VMEM_SHARED((ACC_ROWS,), jnp.float32),
    ] + [pltpu.SemaphoreType.DMA] * CPW,
    compiler_params=pltpu.CompilerParams(use_tc_tiling_on_sc=False),
)
def _sc_counts(edges_hbm, out_hbm, dst_v, ones_v, rep_v, acc_sh, *sems):
    cid = lax.axis_index("c")
    sid = lax.axis_index("s")
    wid = sid * NC + cid
    strip = pl.ds(sid * STRIP, STRIP)
    base = wid * EPW

    zero16 = jnp.zeros((16,), jnp.float32)
    ones16 = jnp.ones((16,), jnp.float32)
    for i in range(STRIP // 16):
        ones_v[pl.ds(i * 16, 16)] = zero16
    pltpu.sync_copy(ones_v.at[pl.ds(0, STRIP)], acc_sh.at[strip])
    for i in range(CHUNK // 16):
        ones_v[pl.ds(i * 16, 16)] = ones16
    if CHUNK % 16:
        ones_v[pl.ds(CHUNK - 16, 16)] = ones16
    for c in range(CPW):
        pltpu.sync_copy(
            edges_hbm.at[1, pl.ds(base + c * CHUNK, CHUNK)], dst_v.at[c])
    plsc.subcore_barrier()

    copies = [
        pltpu.async_copy(ones_v, acc_sh.at[dst_v.at[c]], sems[c], add=True)
        for c in range(CPW)
    ]
    for cp in copies:
        cp.wait()
    plsc.subcore_barrier()

    pltpu.sync_copy(acc_sh.at[strip], ones_v.at[pl.ds(0, STRIP)])

    def rep_body(k, carry):
        c16 = ones_v[pl.ds(k * 16, 16)]
        for l in range(16):
            rep_v[k * 16 + l, :] = jnp.full((16,), c16[l], jnp.float32)
        return carry

    lax.fori_loop(0, STRIP // 16, rep_body, 0)
    pltpu.sync_copy(rep_v, out_hbm.at[cid, strip])


@functools.partial(
    pl.kernel,
    out_type=jax.ShapeDtypeStruct((NC, ACC_ROWS, HID), jnp.float32),
    mesh=_mesh,
    scratch_types=[
        pltpu.VMEM((CPW, CHUNK), jnp.int32),
        pltpu.VMEM((CPW, CHUNK), jnp.int32),
    ] + [pltpu.VMEM((CHUNK, HID), jnp.float32)] * NBUF
      + [pltpu.VMEM_SHARED((ACC_ROWS, HID), jnp.float32)]
      + [pltpu.VMEM_SHARED((ACC_ROWS, HID), jnp.float32)]
      + [pltpu.SemaphoreType.DMA] * (2 * CPW),
    compiler_params=pltpu.CompilerParams(use_tc_tiling_on_sc=False),
)
def _sc_scatter_rows(vals_hbm, edges_hbm, out_hbm, src_v, dst_v, *rest):
    bufs = rest[:NBUF]
    acc_sh = rest[NBUF]
    table_sh = rest[NBUF + 1]
    semg = rest[NBUF + 2:NBUF + 2 + CPW]
    sems = rest[NBUF + 2 + CPW:]
    cid = lax.axis_index("c")
    sid = lax.axis_index("s")
    wid = sid * NC + cid
    strip = pl.ds(sid * STRIP, STRIP)
    base = wid * EPW

    pltpu.sync_copy(vals_hbm.at[strip], table_sh.at[strip])
    zero16 = jnp.zeros((16,), jnp.float32)

    def zero_buf(i, carry):
        bufs[0][i, :] = zero16
        return carry

    lax.fori_loop(0, CHUNK, zero_buf, 0)
    pltpu.sync_copy(bufs[0].at[pl.ds(0, STRIP)], acc_sh.at[strip])
    for c in range(CPW):
        pltpu.sync_copy(
            edges_hbm.at[0, pl.ds(base + c * CHUNK, CHUNK)], src_v.at[c])
        pltpu.sync_copy(
            edges_hbm.at[1, pl.ds(base + c * CHUNK, CHUNK)], dst_v.at[c])
    plsc.subcore_barrier()

    LEAD = NBUF - 2
    gathers = [None] * CPW
    scatters = [None] * CPW
    for c in range(LEAD):
        gathers[c] = pltpu.async_copy(
            table_sh.at[src_v.at[c]], bufs[c % NBUF], semg[c])
    for c in range(CPW):
        gathers[c].wait()
        scatters[c] = pltpu.async_copy(
            bufs[c % NBUF], acc_sh.at[dst_v.at[c]], sems[c], add=True)
        j = c + LEAD
        if j < CPW:
            if j - NBUF >= 0:
                scatters[j - NBUF].wait()
            gathers[j] = pltpu.async_copy(
                table_sh.at[src_v.at[j]], bufs[j % NBUF], semg[j])
    for c in range(CPW - NBUF, CPW):
        if scatters[c] is not None:
            scatters[c].wait()
    plsc.subcore_barrier()
    pltpu.sync_copy(acc_sh.at[strip], out_hbm.at[cid, strip])


def _tc1_body(c_ref, x_ref, w1_ref, g1_ref, dinv_ref):
    dinv = lax.rsqrt(c_ref[0] + c_ref[1] + 1.0)
    dinv_ref[...] = dinv
    h = jnp.dot(x_ref[...], w1_ref[...], preferred_element_type=jnp.float32)
    g1_ref[pl.ds(0, N_NODES), :] = h * dinv[:N_NODES]
    g1_ref[pl.ds(N_NODES, ACC_ROWS - N_NODES), :] = jnp.zeros(
        (ACC_ROWS - N_NODES, HID), jnp.float32)


def _tc2_body(p_ref, g1_ref, dinv_ref, w2_ref, b1_ref, g2_ref):
    dinv = dinv_ref[...]
    s1 = (p_ref[0] + p_ref[1] + g1_ref[...])[:N_NODES]
    out1 = jnp.maximum(s1 * dinv[:N_NODES] + b1_ref[...], 0.0)
    h2 = jnp.dot(out1, w2_ref[...], preferred_element_type=jnp.float32)
    g2_ref[pl.ds(0, N_NODES), :] = h2 * dinv[:N_NODES]
    g2_ref[pl.ds(N_NODES, ACC_ROWS - N_NODES), :] = jnp.zeros(
        (ACC_ROWS - N_NODES, HID), jnp.float32)


def _tc3_body(p_ref, g2_ref, dinv_ref, b2_ref, wfc_ref, bfc_ref, o_ref):
    s2 = (p_ref[0] + p_ref[1] + g2_ref[...])[:N_NODES]
    out2 = jnp.maximum(s2 * dinv_ref[pl.ds(0, N_NODES), :] + b2_ref[...], 0.0)
    pooled = jnp.sum(out2, axis=0, keepdims=True) * (1.0 / N_NODES)
    z = jnp.dot(pooled, wfc_ref[...], preferred_element_type=jnp.float32)
    o_ref[...] = jax.nn.sigmoid(z + bfc_ref[...])


def kernel(x, edge_index, W1, b1, W2, b2, Wfc, bfc):
    edges = edge_index.astype(jnp.int32)

    cnt = _sc_counts(edges)

    g1, dinv = pl.pallas_call(
        _tc1_body,
        out_shape=(
            jax.ShapeDtypeStruct((ACC_ROWS, HID), jnp.float32),
            jax.ShapeDtypeStruct((ACC_ROWS, HID), jnp.float32),
        ),
    )(cnt, x, W1)

    p1 = _sc_scatter_rows(g1, edges)

    g2 = pl.pallas_call(
        _tc2_body,
        out_shape=jax.ShapeDtypeStruct((ACC_ROWS, HID), jnp.float32),
    )(p1, g1, dinv, W2, b1.reshape(1, HID))

    p2 = _sc_scatter_rows(g2, edges)

    out = pl.pallas_call(
        _tc3_body,
        out_shape=jax.ShapeDtypeStruct((1, 1), jnp.float32),
    )(p2, g2, dinv, b2.reshape(1, HID), Wfc, bfc.reshape(1, 1))
    return out.reshape(1)


# speedup vs baseline: 1.4199x; 1.2383x over previous
import functools

import jax
import jax.numpy as jnp
from jax import lax
from jax.experimental import pallas as pl
from jax.experimental.pallas import tpu as pltpu
from jax.experimental.pallas import tpu_sc as plsc

N_NODES = 10000
N_EDGES = 320000
IN_DIM = 128
HID = 16

NC = 2
NS = 16
NW = NC * NS

EPW = N_EDGES // NW
CHUNK = 1000
CPW = EPW // CHUNK
NBUF = 4

ACC_ROWS = 10240
STRIP = ACC_ROWS // NS

_mesh = plsc.VectorSubcoreMesh(core_axis_name="c", subcore_axis_name="s")


@functools.partial(
    pl.kernel,
    out_type=jax.ShapeDtypeStruct((NC, ACC_ROWS, HID), jnp.float32),
    mesh=_mesh,
    scratch_types=[
        pltpu.VMEM((CPW, CHUNK), jnp.int32),
        pltpu.VMEM((CHUNK,), jnp.float32),
        pltpu.VMEM((STRIP, HID), jnp.float32),
        pltpu.VMEM_SHARED((ACC_ROWS,), jnp.float32),
    ] + [pltpu.SemaphoreType.DMA] * CPW,
    compiler_params=pltpu.CompilerParams(use_tc_tiling_on_sc=False),
)
def _sc_counts(edges_hbm, out_hbm, dst_v, ones_v, rep_v, acc_sh, *sems):
    cid = lax.axis_index("c")
    sid = lax.axis_index("s")
    wid = sid * NC + cid
    strip = pl.ds(sid * STRIP, STRIP)
    base = wid * EPW

    zero16 = jnp.zeros((16,), jnp.float32)
    ones16 = jnp.ones((16,), jnp.float32)
    slab_copies = [
        pltpu.async_copy(
            edges_hbm.at[1, pl.ds(base + c * CHUNK, CHUNK)], dst_v.at[c],
            sems[c])
        for c in range(CPW)
    ]
    for i in range(STRIP // 16):
        ones_v[pl.ds(i * 16, 16)] = zero16
    pltpu.sync_copy(ones_v.at[pl.ds(0, STRIP)], acc_sh.at[strip])
    for i in range(CHUNK // 16):
        ones_v[pl.ds(i * 16, 16)] = ones16
    if CHUNK % 16:
        ones_v[pl.ds(CHUNK - 16, 16)] = ones16
    for cp in slab_copies:
        cp.wait()
    plsc.subcore_barrier()

    copies = [
        pltpu.async_copy(ones_v, acc_sh.at[dst_v.at[c]], sems[c], add=True)
        for c in range(CPW)
    ]
    for cp in copies:
        cp.wait()
    plsc.subcore_barrier()

    pltpu.sync_copy(acc_sh.at[strip], ones_v.at[pl.ds(0, STRIP)])

    def rep_body(k, carry):
        c16 = ones_v[pl.ds(k * 16, 16)]
        for l in range(16):
            rep_v[k * 16 + l, :] = jnp.full((16,), c16[l], jnp.float32)
        return carry

    lax.fori_loop(0, STRIP // 16, rep_body, 0)
    pltpu.sync_copy(rep_v, out_hbm.at[cid, strip])


@functools.partial(
    pl.kernel,
    out_type=jax.ShapeDtypeStruct((NC, ACC_ROWS, HID), jnp.float32),
    mesh=_mesh,
    scratch_types=[
        pltpu.VMEM((CPW, CHUNK), jnp.int32),
        pltpu.VMEM((CPW, CHUNK), jnp.int32),
    ] + [pltpu.VMEM((CHUNK, HID), jnp.float32)] * NBUF
      + [pltpu.VMEM_SHARED((ACC_ROWS, HID), jnp.float32)]
      + [pltpu.VMEM_SHARED((ACC_ROWS, HID), jnp.float32)]
      + [pltpu.SemaphoreType.DMA] * (2 * CPW),
    compiler_params=pltpu.CompilerParams(use_tc_tiling_on_sc=False),
)
def _sc_scatter_rows(vals_hbm, edges_hbm, out_hbm, src_v, dst_v, *rest):
    bufs = rest[:NBUF]
    acc_sh = rest[NBUF]
    table_sh = rest[NBUF + 1]
    semg = rest[NBUF + 2:NBUF + 2 + CPW]
    sems = rest[NBUF + 2 + CPW:]
    cid = lax.axis_index("c")
    sid = lax.axis_index("s")
    wid = sid * NC + cid
    strip = pl.ds(sid * STRIP, STRIP)
    base = wid * EPW

    all_sems = list(semg) + list(sems)
    prelude = [pltpu.async_copy(
        vals_hbm.at[strip], table_sh.at[strip], all_sems[0])]
    prelude += [
        pltpu.async_copy(
            edges_hbm.at[d, pl.ds(base + c * CHUNK, CHUNK)],
            (src_v if d == 0 else dst_v).at[c],
            all_sems[(1 + d * CPW + c) % (2 * CPW)])
        for d in range(2) for c in range(CPW)
    ]
    zero16 = jnp.zeros((16,), jnp.float32)

    def zero_buf(i, carry):
        bufs[0][i, :] = zero16
        return carry

    lax.fori_loop(0, CHUNK, zero_buf, 0)
    pltpu.sync_copy(bufs[0].at[pl.ds(0, STRIP)], acc_sh.at[strip])
    for cp in prelude:
        cp.wait()
    plsc.subcore_barrier()

    LEAD = NBUF - 2
    gathers = [None] * CPW
    scatters = [None] * CPW
    for c in range(LEAD):
        gathers[c] = pltpu.async_copy(
            table_sh.at[src_v.at[c]], bufs[c % NBUF], semg[c])
    for c in range(CPW):
        gathers[c].wait()
        scatters[c] = pltpu.async_copy(
            bufs[c % NBUF], acc_sh.at[dst_v.at[c]], sems[c], add=True)
        j = c + LEAD
        if j < CPW:
            if j - NBUF >= 0:
                scatters[j - NBUF].wait()
            gathers[j] = pltpu.async_copy(
                table_sh.at[src_v.at[j]], bufs[j % NBUF], semg[j])
    for c in range(CPW - NBUF, CPW):
        if scatters[c] is not None:
            scatters[c].wait()
    plsc.subcore_barrier()
    pltpu.sync_copy(acc_sh.at[strip], out_hbm.at[cid, strip])


def _tc1_body(c_ref, x_ref, w1_ref, g1_ref, dinv_ref):
    dinv = lax.rsqrt(c_ref[0] + c_ref[1] + 1.0)
    dinv_ref[...] = dinv
    h = jnp.dot(x_ref[...], w1_ref[...], preferred_element_type=jnp.float32)
    g1_ref[pl.ds(0, N_NODES), :] = h * dinv[:N_NODES]
    g1_ref[pl.ds(N_NODES, ACC_ROWS - N_NODES), :] = jnp.zeros(
        (ACC_ROWS - N_NODES, HID), jnp.float32)


def _tc2_body(p_ref, g1_ref, dinv_ref, w2_ref, b1_ref, g2_ref):
    dinv = dinv_ref[...]
    s1 = (p_ref[0] + p_ref[1] + g1_ref[...])[:N_NODES]
    out1 = jnp.maximum(s1 * dinv[:N_NODES] + b1_ref[...], 0.0)
    h2 = jnp.dot(out1, w2_ref[...], preferred_element_type=jnp.float32)
    g2_ref[pl.ds(0, N_NODES), :] = h2 * dinv[:N_NODES]
    g2_ref[pl.ds(N_NODES, ACC_ROWS - N_NODES), :] = jnp.zeros(
        (ACC_ROWS - N_NODES, HID), jnp.float32)


def _tc3_body(p_ref, g2_ref, dinv_ref, b2_ref, wfc_ref, bfc_ref, o_ref):
    s2 = (p_ref[0] + p_ref[1] + g2_ref[...])[:N_NODES]
    out2 = jnp.maximum(s2 * dinv_ref[pl.ds(0, N_NODES), :] + b2_ref[...], 0.0)
    pooled = jnp.sum(out2, axis=0, keepdims=True) * (1.0 / N_NODES)
    z = jnp.dot(pooled, wfc_ref[...], preferred_element_type=jnp.float32)
    o_ref[...] = jax.nn.sigmoid(z + bfc_ref[...])


def kernel(x, edge_index, W1, b1, W2, b2, Wfc, bfc):
    edges = edge_index.astype(jnp.int32)

    cnt = _sc_counts(edges)

    g1, dinv = pl.pallas_call(
        _tc1_body,
        out_shape=(
            jax.ShapeDtypeStruct((ACC_ROWS, HID), jnp.float32),
            jax.ShapeDtypeStruct((ACC_ROWS, HID), jnp.float32),
        ),
    )(cnt, x, W1)

    p1 = _sc_scatter_rows(g1, edges)

    g2 = pl.pallas_call(
        _tc2_body,
        out_shape=jax.ShapeDtypeStruct((ACC_ROWS, HID), jnp.float32),
    )(p1, g1, dinv, W2, b1.reshape(1, HID))

    p2 = _sc_scatter_rows(g2, edges)

    out = pl.pallas_call(
        _tc3_body,
        out_shape=jax.ShapeDtypeStruct((1, 1), jnp.float32),
    )(p2, g2, dinv, b2.reshape(1, HID), Wfc, bfc.reshape(1, 1))
    return out.reshape(1)
